# baseline (speedup 1.0000x reference)
import functools
import jax
import jax.numpy as jnp
from jax import lax
from jax.experimental import pallas as pl
from jax.experimental.pallas import tpu as pltpu
from jax.experimental.pallas import tpu_sc as plsc

N = 10000
E = 320000
NC = 2
NS = 16
NW = NC * NS
CH = 125
NCHUNK = E // NW // CH
NPAD = 10240
STRIPE = NPAD // NS
ZR = 64

_mesh = plsc.VectorSubcoreMesh(core_axis_name="c", subcore_axis_name="s")
_sc_params = pltpu.CompilerParams(use_tc_tiling_on_sc=False)


@functools.partial(
    pl.kernel,
    out_type=jax.ShapeDtypeStruct((NC * NPAD,), jnp.float32),
    mesh=_mesh,
    scratch_types=[
        pltpu.VMEM((NCHUNK, CH), jnp.int32),
        pltpu.VMEM((128,), jnp.float32),
        pltpu.VMEM((STRIPE,), jnp.float32),
        pltpu.SemaphoreType.DMA,
        pltpu.VMEM_SHARED((NPAD,), jnp.float32),
    ],
    compiler_params=_sc_params,
)
def _deg_kernel(dst_hbm, out_hbm, didx, ones, zrow, sem, acc):
    c = lax.axis_index("c")
    s = lax.axis_index("s")
    w = s * NC + c
    one16 = jnp.ones((16,), jnp.float32)
    zero16 = jnp.zeros((16,), jnp.float32)
    for k in range(128 // 16):
        ones[pl.ds(k * 16, 16)] = one16
    for k in range(STRIPE // 16):
        zrow[pl.ds(k * 16, 16)] = zero16
    pltpu.sync_copy(zrow, acc.at[pl.ds(s * STRIPE, STRIPE)])
    pltpu.sync_copy(dst_hbm.at[pl.ds(w * NCHUNK, NCHUNK)], didx)
    plsc.subcore_barrier()

    def body(j, carry):
        ds = [pltpu.async_copy(ones.at[pl.ds(0, CH)],
                               acc.at[didx.at[4 * j + b]], sem, add=True)
              for b in range(4)]
        for d in ds:
            d.wait()
        return carry

    lax.fori_loop(0, NCHUNK // 4, body, 0)
    plsc.subcore_barrier()
    pltpu.sync_copy(acc.at[pl.ds(s * STRIPE, STRIPE)],
                    out_hbm.at[pl.ds(c * NPAD + s * STRIPE, STRIPE)])


K = 8
NGRP = NCHUNK // K


def _make_rowsum(D, stage):
    scratch = [
        pltpu.VMEM((NCHUNK, CH), jnp.int32),
        pltpu.VMEM((NCHUNK, CH), jnp.int32),
        pltpu.VMEM((K, CH, D), jnp.float32),
        pltpu.VMEM((ZR, D), jnp.float32),
        pltpu.SemaphoreType.DMA,
        pltpu.SemaphoreType.DMA,
        pltpu.VMEM_SHARED((NPAD, D), jnp.float32),
    ]
    if stage:
        scratch.append(pltpu.VMEM_SHARED((NPAD, D), jnp.float32))

    @functools.partial(
        pl.kernel,
        out_type=jax.ShapeDtypeStruct((NC * NPAD, D), jnp.float32),
        mesh=_mesh,
        scratch_types=scratch,
        compiler_params=_sc_params,
    )
    def _rowsum(vals_hbm, src_hbm, dst_hbm, out_hbm,
                sidx, didx, rows, zbuf, gsem, ssem, acc, *maybe_svals):
        c = lax.axis_index("c")
        s = lax.axis_index("s")
        w = s * NC + c
        if stage:
            svals = maybe_svals[0]
            for k in range(STRIPE // ZR):
                pltpu.sync_copy(vals_hbm.at[pl.ds(s * STRIPE + k * ZR, ZR)],
                                zbuf)
                pltpu.sync_copy(zbuf, svals.at[pl.ds(s * STRIPE + k * ZR, ZR)])
            gsrc = svals
        else:
            gsrc = vals_hbm
        zero16 = jnp.zeros((16,), jnp.float32)
        for r in range(ZR):
            for k in range(D // 16):
                zbuf[r, pl.ds(k * 16, 16)] = zero16
        for k in range(STRIPE // ZR):
            pltpu.sync_copy(zbuf, acc.at[pl.ds(s * STRIPE + k * ZR, ZR)])
        pltpu.sync_copy(src_hbm.at[pl.ds(w * NCHUNK, NCHUNK)], sidx)
        pltpu.sync_copy(dst_hbm.at[pl.ds(w * NCHUNK, NCHUNK)], didx)
        plsc.subcore_barrier()

        def group(g, carry):
            base = g * K
            gds = [pltpu.async_copy(gsrc.at[sidx.at[base + b]],
                                    rows.at[b], gsem) for b in range(K)]
            sds = []
            for b in range(K):
                gds[b].wait()
                sds.append(pltpu.async_copy(rows.at[b],
                                            acc.at[didx.at[base + b]],
                                            ssem, add=True))
            for d in sds:
                d.wait()
            return carry

        lax.fori_loop(0, NGRP, group, 0)
        plsc.subcore_barrier()
        pltpu.sync_copy(acc.at[pl.ds(s * STRIPE, STRIPE)],
                        out_hbm.at[pl.ds(c * NPAD + s * STRIPE, STRIPE)])

    return _rowsum


_rowsum64 = _make_rowsum(64, stage=False)
_rowsum16 = _make_rowsum(16, stage=False)


def _dinv_from(degp_ref):
    deg = degp_ref[0, :N] + degp_ref[1, :N] + 1.0
    return lax.rsqrt(deg)[:, None]


def _dense_body(x_ref, w1_ref, b1_ref, w2_ref, wc1_ref, b2_ref, p_ref):
    wf = jnp.dot(w2_ref[...], wc1_ref[...], preferred_element_type=jnp.float32)
    h = jax.nn.relu(jnp.dot(x_ref[...], w1_ref[...],
                            preferred_element_type=jnp.float32) + b1_ref[...])
    bf = jnp.dot(b2_ref[...], wc1_ref[...], preferred_element_type=jnp.float32)
    p_ref[...] = jnp.dot(h, wf, preferred_element_type=jnp.float32) + bf


_tc_dense = pl.pallas_call(
    _dense_body,
    out_shape=jax.ShapeDtypeStruct((N, 64), jnp.float32),
)


def _scale_body(p_ref, degp_ref, ps_ref):
    ps_ref[0:N, :] = p_ref[...] * _dinv_from(degp_ref)
    ps_ref[N:NPAD, :] = jnp.zeros((NPAD - N, 64), jnp.float32)


_tc_scale = pl.pallas_call(
    _scale_body,
    out_shape=jax.ShapeDtypeStruct((NPAD, 64), jnp.float32),
)


def _bn_body(sp_ref, ps_ref, degp_ref, gamma_ref, beta_ref, wc2_ref, qp_ref):
    dinv = _dinv_from(degp_ref)
    h1 = dinv * (sp_ref[0:N, :] + sp_ref[NPAD:NPAD + N, :] + ps_ref[0:N, :])
    mean = jnp.mean(h1, axis=0, keepdims=True)
    var = jnp.mean((h1 - mean) ** 2, axis=0, keepdims=True)
    z = jax.nn.relu((h1 - mean) * lax.rsqrt(var + 1e-5) * gamma_ref[...]
                    + beta_ref[...])
    q = jnp.dot(z, wc2_ref[...], preferred_element_type=jnp.float32) * dinv
    qp_ref[0:N, :] = jnp.concatenate(
        [q, jnp.zeros((N, 14), jnp.float32)], axis=1)
    qp_ref[N:NPAD, :] = jnp.zeros((NPAD - N, 16), jnp.float32)


_tc_bn = pl.pallas_call(
    _bn_body,
    out_shape=jax.ShapeDtypeStruct((NPAD, 16), jnp.float32),
)


def _final_body(s2_ref, qp_ref, degp_ref, bc2_ref, out_ref):
    dinv = _dinv_from(degp_ref)
    out_ref[...] = dinv * (s2_ref[0:N, 0:2] + s2_ref[NPAD:NPAD + N, 0:2]
                           + qp_ref[0:N, 0:2]) + bc2_ref[...]


_tc_final = pl.pallas_call(
    _final_body,
    out_shape=jax.ShapeDtypeStruct((N, 2), jnp.float32),
)


@jax.jit
def kernel(x, edge_index, W1, b1, W2, b2, Wc1, bc1, gamma, beta, Wc2, bc2):
    src = edge_index[0].reshape(NW * NCHUNK, CH)
    dst = edge_index[1].reshape(NW * NCHUNK, CH)

    degp = _deg_kernel(dst).reshape(NC, NPAD)
    p = _tc_dense(x, W1, b1.reshape(1, 32), W2, Wc1, b2.reshape(1, 128))
    ps = _tc_scale(p, degp)
    sp = _rowsum64(ps, src, dst)
    qp = _tc_bn(sp, ps, degp, gamma.reshape(1, 64), beta.reshape(1, 64), Wc2)
    s2 = _rowsum16(qp, src, dst)
    return _tc_final(s2, qp, degp, bc2.reshape(1, 2))


# speedup vs baseline: 1.0204x; 1.0204x over previous
import functools
import jax
import jax.numpy as jnp
from jax import lax
from jax.experimental import pallas as pl
from jax.experimental.pallas import tpu as pltpu
from jax.experimental.pallas import tpu_sc as plsc

N = 10000
E = 320000
NC = 2
NS = 16
NW = NC * NS
CH = 125
NCHUNK = E // NW // CH
NPAD = 10240
STRIPE = NPAD // NS
ZR = 64

_mesh = plsc.VectorSubcoreMesh(core_axis_name="c", subcore_axis_name="s")
_sc_params = pltpu.CompilerParams(use_tc_tiling_on_sc=False)


@functools.partial(
    pl.kernel,
    out_type=jax.ShapeDtypeStruct((NC * NPAD,), jnp.float32),
    mesh=_mesh,
    scratch_types=[
        pltpu.VMEM((NCHUNK, CH), jnp.int32),
        pltpu.VMEM((128,), jnp.float32),
        pltpu.VMEM((STRIPE,), jnp.float32),
        pltpu.SemaphoreType.DMA,
        pltpu.VMEM_SHARED((NPAD,), jnp.float32),
    ],
    compiler_params=_sc_params,
)
def _deg_kernel(dst_hbm, out_hbm, didx, ones, zrow, sem, acc):
    c = lax.axis_index("c")
    s = lax.axis_index("s")
    w = s * NC + c
    one16 = jnp.ones((16,), jnp.float32)
    zero16 = jnp.zeros((16,), jnp.float32)
    for k in range(128 // 16):
        ones[pl.ds(k * 16, 16)] = one16
    for k in range(STRIPE // 16):
        zrow[pl.ds(k * 16, 16)] = zero16
    pltpu.sync_copy(zrow, acc.at[pl.ds(s * STRIPE, STRIPE)])
    pltpu.sync_copy(dst_hbm.at[pl.ds(w * NCHUNK, NCHUNK)], didx)
    plsc.subcore_barrier()

    def body(j, carry):
        ds = [pltpu.async_copy(ones.at[pl.ds(0, CH)],
                               acc.at[didx.at[4 * j + b]], sem, add=True)
              for b in range(4)]
        for d in ds:
            d.wait()
        return carry

    lax.fori_loop(0, NCHUNK // 4, body, 0)
    plsc.subcore_barrier()
    pltpu.sync_copy(acc.at[pl.ds(s * STRIPE, STRIPE)],
                    out_hbm.at[pl.ds(c * NPAD + s * STRIPE, STRIPE)])


K = 8
NGRP = NCHUNK // K


def _make_rowsum(D, stage):
    scratch = [
        pltpu.VMEM((NCHUNK, CH), jnp.int32),
        pltpu.VMEM((NCHUNK, CH), jnp.int32),
        pltpu.VMEM((K, CH, D), jnp.float32),
        pltpu.VMEM((ZR, D), jnp.float32),
        pltpu.SemaphoreType.DMA,
        pltpu.SemaphoreType.DMA,
        pltpu.VMEM_SHARED((NPAD, D), jnp.float32),
    ]
    if stage:
        scratch.append(pltpu.VMEM_SHARED((NPAD, D), jnp.float32))

    @functools.partial(
        pl.kernel,
        out_type=jax.ShapeDtypeStruct((NC * NPAD, D), jnp.float32),
        mesh=_mesh,
        scratch_types=scratch,
        compiler_params=_sc_params,
    )
    def _rowsum(vals_hbm, src_hbm, dst_hbm, out_hbm,
                sidx, didx, rows, zbuf, gsem, ssem, acc, *maybe_svals):
        c = lax.axis_index("c")
        s = lax.axis_index("s")
        w = s * NC + c
        if stage:
            svals = maybe_svals[0]
            for k in range(STRIPE // ZR):
                pltpu.sync_copy(vals_hbm.at[pl.ds(s * STRIPE + k * ZR, ZR)],
                                zbuf)
                pltpu.sync_copy(zbuf, svals.at[pl.ds(s * STRIPE + k * ZR, ZR)])
            gsrc = svals
        else:
            gsrc = vals_hbm
        zero16 = jnp.zeros((16,), jnp.float32)
        for r in range(ZR):
            for k in range(D // 16):
                zbuf[r, pl.ds(k * 16, 16)] = zero16
        for k in range(STRIPE // ZR):
            pltpu.sync_copy(zbuf, acc.at[pl.ds(s * STRIPE + k * ZR, ZR)])
        pltpu.sync_copy(src_hbm.at[pl.ds(w * NCHUNK, NCHUNK)], sidx)
        pltpu.sync_copy(dst_hbm.at[pl.ds(w * NCHUNK, NCHUNK)], didx)
        plsc.subcore_barrier()

        def group(g, carry):
            base = g * K
            gds = [pltpu.async_copy(gsrc.at[sidx.at[base + b]],
                                    rows.at[b], gsem) for b in range(K)]
            sds = []
            for b in range(K):
                gds[b].wait()
                sds.append(pltpu.async_copy(rows.at[b],
                                            acc.at[didx.at[base + b]],
                                            ssem, add=True))
            for d in sds:
                d.wait()
            return carry

        lax.fori_loop(0, NGRP, group, 0)
        plsc.subcore_barrier()
        pltpu.sync_copy(acc.at[pl.ds(s * STRIPE, STRIPE)],
                        out_hbm.at[pl.ds(c * NPAD + s * STRIPE, STRIPE)])

    return _rowsum


_rowsum64 = _make_rowsum(64, stage=False)
_rowsum16 = _make_rowsum(16, stage=False)


def _dinv_from(degp_ref):
    deg = degp_ref[0, :N] + degp_ref[1, :N] + 1.0
    return lax.rsqrt(deg)[:, None]


def _dense_body(x_ref, w1_ref, b1_ref, w2_ref, wc1_ref, b2_ref, degp_ref,
                ps_ref):
    wf = jnp.dot(w2_ref[...], wc1_ref[...], preferred_element_type=jnp.float32)
    h = jax.nn.relu(jnp.dot(x_ref[...], w1_ref[...],
                            preferred_element_type=jnp.float32) + b1_ref[...])
    bf = jnp.dot(b2_ref[...], wc1_ref[...], preferred_element_type=jnp.float32)
    p = jnp.dot(h, wf, preferred_element_type=jnp.float32) + bf
    ps_ref[0:N, :] = p * _dinv_from(degp_ref)
    ps_ref[N:NPAD, :] = jnp.zeros((NPAD - N, 64), jnp.float32)


_tc_dense = pl.pallas_call(
    _dense_body,
    out_shape=jax.ShapeDtypeStruct((NPAD, 64), jnp.float32),
)


def _bn_body(sp_ref, ps_ref, degp_ref, gamma_ref, beta_ref, wc2_ref, qp_ref):
    dinv = _dinv_from(degp_ref)
    h1 = dinv * (sp_ref[0:N, :] + sp_ref[NPAD:NPAD + N, :] + ps_ref[0:N, :])
    mean = jnp.mean(h1, axis=0, keepdims=True)
    var = jnp.mean((h1 - mean) ** 2, axis=0, keepdims=True)
    z = jax.nn.relu((h1 - mean) * lax.rsqrt(var + 1e-5) * gamma_ref[...]
                    + beta_ref[...])
    q = jnp.dot(z, wc2_ref[...], preferred_element_type=jnp.float32) * dinv
    qp_ref[0:N, :] = jnp.concatenate(
        [q, jnp.zeros((N, 14), jnp.float32)], axis=1)
    qp_ref[N:NPAD, :] = jnp.zeros((NPAD - N, 16), jnp.float32)


_tc_bn = pl.pallas_call(
    _bn_body,
    out_shape=jax.ShapeDtypeStruct((NPAD, 16), jnp.float32),
)


def _final_body(s2_ref, qp_ref, degp_ref, bc2_ref, out_ref):
    dinv = _dinv_from(degp_ref)
    out_ref[...] = dinv * (s2_ref[0:N, 0:2] + s2_ref[NPAD:NPAD + N, 0:2]
                           + qp_ref[0:N, 0:2]) + bc2_ref[...]


_tc_final = pl.pallas_call(
    _final_body,
    out_shape=jax.ShapeDtypeStruct((N, 2), jnp.float32),
)


@jax.jit
def kernel(x, edge_index, W1, b1, W2, b2, Wc1, bc1, gamma, beta, Wc2, bc2):
    src = edge_index[0].reshape(NW * NCHUNK, CH)
    dst = edge_index[1].reshape(NW * NCHUNK, CH)

    degp = _deg_kernel(dst).reshape(NC, NPAD)
    ps = _tc_dense(x, W1, b1.reshape(1, 32), W2, Wc1, b2.reshape(1, 128),
                   degp)
    sp = _rowsum64(ps, src, dst)
    qp = _tc_bn(sp, ps, degp, gamma.reshape(1, 64), beta.reshape(1, 64), Wc2)
    s2 = _rowsum16(qp, src, dst)
    return _tc_final(s2, qp, degp, bc2.reshape(1, 2))


# speedup vs baseline: 1.0223x; 1.0019x over previous
import functools
import jax
import jax.numpy as jnp
from jax import lax
from jax.experimental import pallas as pl
from jax.experimental.pallas import tpu as pltpu
from jax.experimental.pallas import tpu_sc as plsc

N = 10000
E = 320000
NC = 2
NS = 16
NW = NC * NS
CH = 125
NCHUNK = E // NW // CH
NPAD = 10240
STRIPE = NPAD // NS
ZR = 64

_mesh = plsc.VectorSubcoreMesh(core_axis_name="c", subcore_axis_name="s")
_sc_params = pltpu.CompilerParams(use_tc_tiling_on_sc=False)


@functools.partial(
    pl.kernel,
    out_type=jax.ShapeDtypeStruct((NC * NPAD,), jnp.float32),
    mesh=_mesh,
    scratch_types=[
        pltpu.VMEM((NCHUNK, CH), jnp.int32),
        pltpu.VMEM((128,), jnp.float32),
        pltpu.VMEM((STRIPE,), jnp.float32),
        pltpu.SemaphoreType.DMA,
        pltpu.VMEM_SHARED((NPAD,), jnp.float32),
    ],
    compiler_params=_sc_params,
)
def _deg_kernel(dst_hbm, out_hbm, didx, ones, zrow, sem, acc):
    c = lax.axis_index("c")
    s = lax.axis_index("s")
    w = s * NC + c
    one16 = jnp.ones((16,), jnp.float32)
    zero16 = jnp.zeros((16,), jnp.float32)
    for k in range(128 // 16):
        ones[pl.ds(k * 16, 16)] = one16
    for k in range(STRIPE // 16):
        zrow[pl.ds(k * 16, 16)] = zero16
    pltpu.sync_copy(zrow, acc.at[pl.ds(s * STRIPE, STRIPE)])
    pltpu.sync_copy(dst_hbm.at[pl.ds(w * NCHUNK, NCHUNK)], didx)
    plsc.subcore_barrier()

    def body(j, carry):
        ds = [pltpu.async_copy(ones.at[pl.ds(0, CH)],
                               acc.at[didx.at[4 * j + b]], sem, add=True)
              for b in range(4)]
        for d in ds:
            d.wait()
        return carry

    lax.fori_loop(0, NCHUNK // 4, body, 0)
    plsc.subcore_barrier()
    pltpu.sync_copy(acc.at[pl.ds(s * STRIPE, STRIPE)],
                    out_hbm.at[pl.ds(c * NPAD + s * STRIPE, STRIPE)])


K = 8
NGRP = NCHUNK // K


def _make_rowsum(D, stage):
    scratch = [
        pltpu.VMEM((NCHUNK, CH), jnp.int32),
        pltpu.VMEM((NCHUNK, CH), jnp.int32),
        pltpu.VMEM((K, CH, D), jnp.float32),
        pltpu.VMEM((ZR, D), jnp.float32),
        pltpu.SemaphoreType.DMA,
        pltpu.SemaphoreType.DMA,
        pltpu.VMEM_SHARED((NPAD, D), jnp.float32),
    ]
    if stage:
        scratch.append(pltpu.VMEM_SHARED((NPAD, D), jnp.float32))

    @functools.partial(
        pl.kernel,
        out_type=jax.ShapeDtypeStruct((NC * NPAD, D), jnp.float32),
        mesh=_mesh,
        scratch_types=scratch,
        compiler_params=_sc_params,
    )
    def _rowsum(vals_hbm, src_hbm, dst_hbm, out_hbm,
                sidx, didx, rows, zbuf, gsem, ssem, acc, *maybe_svals):
        c = lax.axis_index("c")
        s = lax.axis_index("s")
        w = s * NC + c
        if stage:
            svals = maybe_svals[0]
            for k in range(STRIPE // ZR):
                pltpu.sync_copy(vals_hbm.at[pl.ds(s * STRIPE + k * ZR, ZR)],
                                zbuf)
                pltpu.sync_copy(zbuf, svals.at[pl.ds(s * STRIPE + k * ZR, ZR)])
            gsrc = svals
        else:
            gsrc = vals_hbm
        zero16 = jnp.zeros((16,), jnp.float32)
        for r in range(ZR):
            for k in range(D // 16):
                zbuf[r, pl.ds(k * 16, 16)] = zero16
        for k in range(STRIPE // ZR):
            pltpu.sync_copy(zbuf, acc.at[pl.ds(s * STRIPE + k * ZR, ZR)])
        pltpu.sync_copy(src_hbm.at[pl.ds(w * NCHUNK, NCHUNK)], sidx)
        pltpu.sync_copy(dst_hbm.at[pl.ds(w * NCHUNK, NCHUNK)], didx)
        plsc.subcore_barrier()

        def group(g, carry):
            base = g * K
            gds = [pltpu.async_copy(gsrc.at[sidx.at[base + b]],
                                    rows.at[b], gsem) for b in range(K)]
            sds = []
            for b in range(K):
                gds[b].wait()
                sds.append(pltpu.async_copy(rows.at[b],
                                            acc.at[didx.at[base + b]],
                                            ssem, add=True))
            for d in sds:
                d.wait()
            return carry

        lax.fori_loop(0, NGRP, group, 0)
        plsc.subcore_barrier()
        pltpu.sync_copy(acc.at[pl.ds(s * STRIPE, STRIPE)],
                        out_hbm.at[pl.ds(c * NPAD + s * STRIPE, STRIPE)])

    return _rowsum


_rowsum64 = _make_rowsum(64, stage=False)
_rowsum16 = _make_rowsum(16, stage=False)


def _dinv_from(degp_ref):
    deg = degp_ref[0, :N] + degp_ref[1, :N] + 1.0
    return lax.rsqrt(deg)[:, None]


def _dense_body(x_ref, w1_ref, b1_ref, w2_ref, wc1_ref, b2_ref, degp_ref,
                ps_ref):
    wf = jnp.dot(w2_ref[...], wc1_ref[...], preferred_element_type=jnp.float32)
    h = jax.nn.relu(jnp.dot(x_ref[...], w1_ref[...],
                            preferred_element_type=jnp.float32) + b1_ref[...])
    bf = jnp.dot(b2_ref[...], wc1_ref[...], preferred_element_type=jnp.float32)
    p = jnp.dot(h, wf, preferred_element_type=jnp.float32) + bf
    ps_ref[0:N, :] = p * _dinv_from(degp_ref)
    ps_ref[N:NPAD, :] = jnp.zeros((NPAD - N, 64), jnp.float32)


_tc_dense = pl.pallas_call(
    _dense_body,
    out_shape=jax.ShapeDtypeStruct((NPAD, 64), jnp.float32),
)


def _bn_body(sp_ref, ps_ref, degp_ref, gamma_ref, beta_ref, wc2_ref, qp_ref):
    dinv = _dinv_from(degp_ref)
    h1 = dinv * (sp_ref[0:N, :] + sp_ref[NPAD:NPAD + N, :] + ps_ref[0:N, :])
    ones_row = jnp.ones((1, N), jnp.float32)
    mean = jnp.dot(ones_row, h1, preferred_element_type=jnp.float32) / N
    d = h1 - mean
    var = jnp.dot(ones_row, d * d, preferred_element_type=jnp.float32) / N
    z = jax.nn.relu((h1 - mean) * lax.rsqrt(var + 1e-5) * gamma_ref[...]
                    + beta_ref[...])
    q = jnp.dot(z, wc2_ref[...], preferred_element_type=jnp.float32) * dinv
    qp_ref[0:N, :] = jnp.concatenate(
        [q, jnp.zeros((N, 14), jnp.float32)], axis=1)
    qp_ref[N:NPAD, :] = jnp.zeros((NPAD - N, 16), jnp.float32)


_tc_bn = pl.pallas_call(
    _bn_body,
    out_shape=jax.ShapeDtypeStruct((NPAD, 16), jnp.float32),
)


def _final_body(s2_ref, qp_ref, degp_ref, bc2_ref, out_ref):
    dinv = _dinv_from(degp_ref)
    out_ref[...] = dinv * (s2_ref[0:N, 0:2] + s2_ref[NPAD:NPAD + N, 0:2]
                           + qp_ref[0:N, 0:2]) + bc2_ref[...]


_tc_final = pl.pallas_call(
    _final_body,
    out_shape=jax.ShapeDtypeStruct((N, 2), jnp.float32),
)


@jax.jit
def kernel(x, edge_index, W1, b1, W2, b2, Wc1, bc1, gamma, beta, Wc2, bc2):
    src = edge_index[0].reshape(NW * NCHUNK, CH)
    dst = edge_index[1].reshape(NW * NCHUNK, CH)

    degp = _deg_kernel(dst).reshape(NC, NPAD)
    ps = _tc_dense(x, W1, b1.reshape(1, 32), W2, Wc1, b2.reshape(1, 128),
                   degp)
    sp = _rowsum64(ps, src, dst)
    qp = _tc_bn(sp, ps, degp, gamma.reshape(1, 64), beta.reshape(1, 64), Wc2)
    s2 = _rowsum16(qp, src, dst)
    return _tc_final(s2, qp, degp, bc2.reshape(1, 2))


# speedup vs baseline: 1.0314x; 1.0089x over previous
import functools
import jax
import jax.numpy as jnp
from jax import lax
from jax.experimental import pallas as pl
from jax.experimental.pallas import tpu as pltpu
from jax.experimental.pallas import tpu_sc as plsc

N = 10000
E = 320000
NC = 2
NS = 16
NW = NC * NS
CH = 125
NCHUNK = E // NW // CH
NPAD = 10240
STRIPE = NPAD // NS
ZR = 64

_mesh = plsc.VectorSubcoreMesh(core_axis_name="c", subcore_axis_name="s")
_sc_params = pltpu.CompilerParams(use_tc_tiling_on_sc=False)


@functools.partial(
    pl.kernel,
    out_type=jax.ShapeDtypeStruct((NC * NPAD,), jnp.float32),
    mesh=_mesh,
    scratch_types=[
        pltpu.VMEM((NCHUNK, CH), jnp.int32),
        pltpu.VMEM((128,), jnp.float32),
        pltpu.VMEM((STRIPE,), jnp.float32),
        pltpu.SemaphoreType.DMA,
        pltpu.VMEM_SHARED((NPAD,), jnp.float32),
    ],
    compiler_params=_sc_params,
)
def _deg_kernel(dst_hbm, out_hbm, didx, ones, zrow, sem, acc):
    c = lax.axis_index("c")
    s = lax.axis_index("s")
    w = s * NC + c
    one16 = jnp.ones((16,), jnp.float32)
    zero16 = jnp.zeros((16,), jnp.float32)
    for k in range(128 // 16):
        ones[pl.ds(k * 16, 16)] = one16
    for k in range(STRIPE // 16):
        zrow[pl.ds(k * 16, 16)] = zero16
    pltpu.sync_copy(zrow, acc.at[pl.ds(s * STRIPE, STRIPE)])
    pltpu.sync_copy(dst_hbm.at[pl.ds(w * NCHUNK, NCHUNK)], didx)
    plsc.subcore_barrier()

    def body(j, carry):
        ds = [pltpu.async_copy(ones.at[pl.ds(0, CH)],
                               acc.at[didx.at[4 * j + b]], sem, add=True)
              for b in range(4)]
        for d in ds:
            d.wait()
        return carry

    lax.fori_loop(0, NCHUNK // 4, body, 0)
    plsc.subcore_barrier()
    pltpu.sync_copy(acc.at[pl.ds(s * STRIPE, STRIPE)],
                    out_hbm.at[pl.ds(c * NPAD + s * STRIPE, STRIPE)])


K = 8
NGRP = NCHUNK // K


def _make_rowsum(D, stage, ch=CH, nchunk=NCHUNK, k=K):
    ngrp = nchunk // k
    scratch = [
        pltpu.VMEM((nchunk, ch), jnp.int32),
        pltpu.VMEM((nchunk, ch), jnp.int32),
        pltpu.VMEM((k, ch, D), jnp.float32),
        pltpu.VMEM((ZR, D), jnp.float32),
        pltpu.SemaphoreType.DMA,
        pltpu.SemaphoreType.DMA,
        pltpu.VMEM_SHARED((NPAD, D), jnp.float32),
    ]
    if stage:
        scratch.append(pltpu.VMEM_SHARED((NPAD, D), jnp.float32))

    @functools.partial(
        pl.kernel,
        out_type=jax.ShapeDtypeStruct((NC * NPAD, D), jnp.float32),
        mesh=_mesh,
        scratch_types=scratch,
        compiler_params=_sc_params,
    )
    def _rowsum(vals_hbm, src_hbm, dst_hbm, out_hbm,
                sidx, didx, rows, zbuf, gsem, ssem, acc, *maybe_svals):
        c = lax.axis_index("c")
        s = lax.axis_index("s")
        w = s * NC + c
        if stage:
            svals = maybe_svals[0]
            for i in range(STRIPE // ZR):
                pltpu.sync_copy(vals_hbm.at[pl.ds(s * STRIPE + i * ZR, ZR)],
                                zbuf)
                pltpu.sync_copy(zbuf, svals.at[pl.ds(s * STRIPE + i * ZR, ZR)])
            gsrc = svals
        else:
            gsrc = vals_hbm
        zero16 = jnp.zeros((16,), jnp.float32)
        for r in range(ZR):
            for i in range(D // 16):
                zbuf[r, pl.ds(i * 16, 16)] = zero16
        for i in range(STRIPE // ZR):
            pltpu.sync_copy(zbuf, acc.at[pl.ds(s * STRIPE + i * ZR, ZR)])
        pltpu.sync_copy(src_hbm.at[pl.ds(w * nchunk, nchunk)], sidx)
        pltpu.sync_copy(dst_hbm.at[pl.ds(w * nchunk, nchunk)], didx)
        plsc.subcore_barrier()

        def group(g, carry):
            base = g * k
            gds = [pltpu.async_copy(gsrc.at[sidx.at[base + b]],
                                    rows.at[b], gsem) for b in range(k)]
            sds = []
            for b in range(k):
                gds[b].wait()
                sds.append(pltpu.async_copy(rows.at[b],
                                            acc.at[didx.at[base + b]],
                                            ssem, add=True))
            for d in sds:
                d.wait()
            return carry

        lax.fori_loop(0, ngrp, group, 0)
        plsc.subcore_barrier()
        pltpu.sync_copy(acc.at[pl.ds(s * STRIPE, STRIPE)],
                        out_hbm.at[pl.ds(c * NPAD + s * STRIPE, STRIPE)])

    return _rowsum


_CH64, _NCH64, _K64 = 250, 40, 4
_CH16, _NCH16, _K16 = 625, 16, 8
_rowsum64 = _make_rowsum(64, stage=False, ch=_CH64, nchunk=_NCH64, k=_K64)
_rowsum16 = _make_rowsum(16, stage=False, ch=_CH16, nchunk=_NCH16, k=_K16)


def _dinv_from(degp_ref):
    deg = degp_ref[0, :N] + degp_ref[1, :N] + 1.0
    return lax.rsqrt(deg)[:, None]


def _dense_body(x_ref, w1_ref, b1_ref, w2_ref, wc1_ref, b2_ref, degp_ref,
                ps_ref):
    wf = jnp.dot(w2_ref[...], wc1_ref[...], preferred_element_type=jnp.float32)
    h = jax.nn.relu(jnp.dot(x_ref[...], w1_ref[...],
                            preferred_element_type=jnp.float32) + b1_ref[...])
    bf = jnp.dot(b2_ref[...], wc1_ref[...], preferred_element_type=jnp.float32)
    p = jnp.dot(h, wf, preferred_element_type=jnp.float32) + bf
    ps_ref[0:N, :] = p * _dinv_from(degp_ref)
    ps_ref[N:NPAD, :] = jnp.zeros((NPAD - N, 64), jnp.float32)


_tc_dense = pl.pallas_call(
    _dense_body,
    out_shape=jax.ShapeDtypeStruct((NPAD, 64), jnp.float32),
)


def _bn_body(sp_ref, ps_ref, degp_ref, gamma_ref, beta_ref, wc2_ref, qp_ref):
    dinv = _dinv_from(degp_ref)
    h1 = dinv * (sp_ref[0:N, :] + sp_ref[NPAD:NPAD + N, :] + ps_ref[0:N, :])
    ones_row = jnp.ones((1, N), jnp.float32)
    mean = jnp.dot(ones_row, h1, preferred_element_type=jnp.float32) / N
    d = h1 - mean
    var = jnp.dot(ones_row, d * d, preferred_element_type=jnp.float32) / N
    z = jax.nn.relu((h1 - mean) * lax.rsqrt(var + 1e-5) * gamma_ref[...]
                    + beta_ref[...])
    q = jnp.dot(z, wc2_ref[...], preferred_element_type=jnp.float32) * dinv
    qp_ref[0:N, :] = jnp.concatenate(
        [q, jnp.zeros((N, 14), jnp.float32)], axis=1)
    qp_ref[N:NPAD, :] = jnp.zeros((NPAD - N, 16), jnp.float32)


_tc_bn = pl.pallas_call(
    _bn_body,
    out_shape=jax.ShapeDtypeStruct((NPAD, 16), jnp.float32),
)


def _final_body(s2_ref, qp_ref, degp_ref, bc2_ref, out_ref):
    dinv = _dinv_from(degp_ref)
    out_ref[...] = dinv * (s2_ref[0:N, 0:2] + s2_ref[NPAD:NPAD + N, 0:2]
                           + qp_ref[0:N, 0:2]) + bc2_ref[...]


_tc_final = pl.pallas_call(
    _final_body,
    out_shape=jax.ShapeDtypeStruct((N, 2), jnp.float32),
)


@jax.jit
def kernel(x, edge_index, W1, b1, W2, b2, Wc1, bc1, gamma, beta, Wc2, bc2):
    src = edge_index[0].reshape(NW * NCHUNK, CH)
    dst = edge_index[1].reshape(NW * NCHUNK, CH)
    src64 = edge_index[0].reshape(NW * _NCH64, _CH64)
    dst64 = edge_index[1].reshape(NW * _NCH64, _CH64)
    src16 = edge_index[0].reshape(NW * _NCH16, _CH16)
    dst16 = edge_index[1].reshape(NW * _NCH16, _CH16)

    degp = _deg_kernel(dst).reshape(NC, NPAD)
    ps = _tc_dense(x, W1, b1.reshape(1, 32), W2, Wc1, b2.reshape(1, 128),
                   degp)
    sp = _rowsum64(ps, src64, dst64)
    qp = _tc_bn(sp, ps, degp, gamma.reshape(1, 64), beta.reshape(1, 64), Wc2)
    s2 = _rowsum16(qp, src16, dst16)
    return _tc_final(s2, qp, degp, bc2.reshape(1, 2))
